# baseline (speedup 1.0000x reference)
import functools

import jax
import jax.numpy as jnp
from jax import lax
from jax.experimental import pallas as pl
from jax.experimental.pallas import tpu as pltpu
from jax.experimental.pallas import tpu_sc as plsc

VOCAB = 100000
DIM = 64
BATCH = 1024
CTX = 20

NUM_CORES = 2
NUM_SUBCORES = 16
LANES = 16
NW = NUM_CORES * NUM_SUBCORES
BPW = BATCH // NW


def _pool_body(x_hbm, emb_hbm, h_hbm, idx_v, rows_v, h_v, sem):
    wid = lax.axis_index("s") * NUM_CORES + lax.axis_index("c")
    pltpu.sync_copy(x_hbm.at[wid], idx_v)
    descs = []
    for e in range(BPW):
        descs.append(pltpu.async_copy(emb_hbm.at[idx_v.at[e]], rows_v.at[e], sem))
    for d in descs:
        d.wait()
    scale = jnp.float32(1.0 / CTX)
    for e in range(BPW):
        def body(j, acc):
            return tuple(
                acc[k] + rows_v[e, j, pl.ds(k * LANES, LANES)] for k in range(DIM // LANES)
            )
        acc0 = tuple(jnp.zeros((LANES,), jnp.float32) for _ in range(DIM // LANES))
        acc = lax.fori_loop(0, CTX, body, acc0)
        for k in range(DIM // LANES):
            h_v[e, pl.ds(k * LANES, LANES)] = acc[k] * scale
    pltpu.sync_copy(h_v, h_hbm.at[pl.ds(wid * BPW, BPW)])


@jax.jit
def _pool(x3, emb):
    mesh = plsc.VectorSubcoreMesh(core_axis_name="c", subcore_axis_name="s")
    return pl.kernel(
        _pool_body,
        out_type=jax.ShapeDtypeStruct((BATCH, DIM), jnp.float32),
        mesh=mesh,
        scratch_types=[
            pltpu.VMEM((BPW, CTX), jnp.int32),
            pltpu.VMEM((BPW, CTX, DIM), jnp.float32),
            pltpu.VMEM((BPW, DIM), jnp.float32),
            pltpu.SemaphoreType.DMA,
        ],
        compiler_params=pltpu.CompilerParams(use_tc_tiling_on_sc=False),
    )(x3, emb)


VT = 512


def _proj_body(h_ref, w_ref, o_ref):
    o_ref[...] = lax.dot_general(
        h_ref[...], w_ref[...],
        dimension_numbers=(((1,), (1,)), ((), ())),
        preferred_element_type=jnp.float32,
    )


@jax.jit
def _project(h, W):
    grid = pl.cdiv(VOCAB, VT)
    return pl.pallas_call(
        _proj_body,
        grid=(grid,),
        in_specs=[
            pl.BlockSpec((BATCH, DIM), lambda i: (0, 0)),
            pl.BlockSpec((VT, DIM), lambda i: (i, 0)),
        ],
        out_specs=pl.BlockSpec((BATCH, VT), lambda i: (0, i)),
        out_shape=jax.ShapeDtypeStruct((BATCH, VOCAB), jnp.float32),
        compiler_params=pltpu.CompilerParams(
            dimension_semantics=("parallel",),
        ),
    )(h, W)


def kernel(x, emb, W):
    x3 = x.reshape(NW, BPW, CTX).astype(jnp.int32)
    h = _pool(x3, emb)
    return _project(h, W)


# speedup vs baseline: 1.1331x; 1.1331x over previous
import functools

import jax
import jax.numpy as jnp
from jax import lax
from jax.experimental import pallas as pl
from jax.experimental.pallas import tpu as pltpu
from jax.experimental.pallas import tpu_sc as plsc

VOCAB = 100000
DIM = 64
BATCH = 1024
CTX = 20

NUM_CORES = 2
NUM_SUBCORES = 16
LANES = 16
NW = NUM_CORES * NUM_SUBCORES
BPW = BATCH // NW
TPW = BPW * CTX
HP = 128
GCH = 128
NCH = TPW // GCH


def _pool_body(pair_hbm, off_hbm, emb2_hbm, h_hbm, idxp_v, offs_v, rows_v, h_v, sem):
    wid = lax.axis_index("s") * NUM_CORES + lax.axis_index("c")
    base = wid * TPW
    pltpu.sync_copy(pair_hbm.at[pl.ds(base, TPW)], idxp_v.at[pl.ds(0, TPW)])
    pltpu.sync_copy(off_hbm.at[pl.ds(base, TPW)], offs_v.at[pl.ds(0, TPW)])
    descs = []
    for c in range(NCH):
        descs.append(
            pltpu.async_copy(
                emb2_hbm.at[idxp_v.at[pl.ds(c * GCH, GCH)]],
                rows_v.at[pl.ds(c * GCH, GCH)],
                sem,
            )
        )
    for d in descs:
        d.wait()
    scale = jnp.float32(1.0 / CTX)
    zeros = jnp.zeros((LANES,), jnp.float32)

    def elem(e, carry):
        accs = [zeros] * (DIM // LANES)
        for j in range(CTX):
            t = e * CTX + j
            off = offs_v[pl.ds(t, LANES)][0]
            for k in range(DIM // LANES):
                accs[k] = accs[k] + rows_v[t, pl.ds(off + k * LANES, LANES)]
        for k in range(DIM // LANES):
            h_v[e, pl.ds(k * LANES, LANES)] = accs[k] * scale
        for k in range(DIM // LANES, HP // LANES):
            h_v[e, pl.ds(k * LANES, LANES)] = zeros
        return carry

    lax.fori_loop(0, BPW, elem, 0)
    pltpu.sync_copy(h_v, h_hbm.at[pl.ds(wid * BPW, BPW)])


@jax.jit
def _pool(pair_flat, off_flat, emb2):
    mesh = plsc.VectorSubcoreMesh(core_axis_name="c", subcore_axis_name="s")
    return pl.kernel(
        _pool_body,
        out_type=jax.ShapeDtypeStruct((BATCH, HP), jnp.float32),
        mesh=mesh,
        scratch_types=[
            pltpu.VMEM((TPW + LANES,), jnp.int32),
            pltpu.VMEM((TPW + LANES,), jnp.int32),
            pltpu.VMEM((TPW, HP), jnp.float32),
            pltpu.VMEM((BPW, HP), jnp.float32),
            pltpu.SemaphoreType.DMA,
        ],
    )(pair_flat, off_flat, emb2)


VT = 2048


def _proj_body(h_ref, w_ref, o_ref):
    o_ref[...] = lax.dot_general(
        h_ref[:, :DIM], w_ref[...],
        dimension_numbers=(((1,), (1,)), ((), ())),
        preferred_element_type=jnp.float32,
    )


@jax.jit
def _project(h, W):
    grid = pl.cdiv(VOCAB, VT)
    return pl.pallas_call(
        _proj_body,
        grid=(grid,),
        in_specs=[
            pl.BlockSpec((BATCH, HP), lambda i: (0, 0)),
            pl.BlockSpec((VT, DIM), lambda i: (i, 0)),
        ],
        out_specs=pl.BlockSpec((BATCH, VT), lambda i: (0, i)),
        out_shape=jax.ShapeDtypeStruct((BATCH, VOCAB), jnp.float32),
        compiler_params=pltpu.CompilerParams(
            dimension_semantics=("parallel",),
        ),
    )(h, W)


def kernel(x, emb, W):
    xi = x.reshape(-1).astype(jnp.int32)
    pair_flat = xi >> 1
    off_flat = (xi & 1) * DIM
    emb2 = emb.reshape(VOCAB // 2, 2 * DIM)
    h = _pool(pair_flat, off_flat, emb2)
    return _project(h, W)
